# baseline (speedup 1.0000x reference)
import jax
import jax.numpy as jnp
from jax.experimental import pallas as pl
from jax.experimental.pallas import tpu as pltpu

BATCH = 16384
K = 1000
L = 16
KB = 40
STEPS = K // KB


def _body(uft_ref, ift_ref, uwt_ref, iwt_ref, out_ref, ul_acc, il_acc):
    i = pl.program_id(0)
    ul_p = jnp.dot(uwt_ref[0], uft_ref[...], preferred_element_type=jnp.float32)
    il_p = jnp.dot(iwt_ref[0], ift_ref[...], preferred_element_type=jnp.float32)

    @pl.when(i == 0)
    def _init():
        ul_acc[...] = ul_p
        il_acc[...] = il_p

    @pl.when(i > 0)
    def _accum():
        ul_acc[...] += ul_p
        il_acc[...] += il_p

    @pl.when(i == STEPS - 1)
    def _finish():
        ul = ul_acc[...]
        il = il_acc[...]
        out_ref[...] = jnp.sum(ul * il[:L], axis=0) + il[L]


def kernel(user_features, item_features, user_latent_w, item_latent_w, item_biases_w):
    uft = user_features.T
    ift = item_features.T
    uwt = user_latent_w.T.reshape(L, STEPS, KB).transpose(1, 0, 2)
    iwt = (
        jnp.concatenate([item_latent_w.T, item_biases_w.T], axis=0)
        .reshape(L + 1, STEPS, KB)
        .transpose(1, 0, 2)
    )
    return pl.pallas_call(
        _body,
        grid=(STEPS,),
        in_specs=[
            pl.BlockSpec((KB, BATCH), lambda i: (i, 0)),
            pl.BlockSpec((KB, BATCH), lambda i: (i, 0)),
            pl.BlockSpec((1, L, KB), lambda i: (i, 0, 0)),
            pl.BlockSpec((1, L + 1, KB), lambda i: (i, 0, 0)),
        ],
        out_specs=pl.BlockSpec((BATCH,), lambda i: (0,)),
        out_shape=jax.ShapeDtypeStruct((BATCH,), jnp.float32),
        scratch_shapes=[
            pltpu.VMEM((L, BATCH), jnp.float32),
            pltpu.VMEM((L + 1, BATCH), jnp.float32),
        ],
    )(uft, ift, uwt, iwt)


# speedup vs baseline: 1.0183x; 1.0183x over previous
import jax
import jax.numpy as jnp
from jax.experimental import pallas as pl
from jax.experimental.pallas import tpu as pltpu

BATCH = 16384
K = 1000
L = 16
KB = 200
STEPS = K // KB


def _body(uft_ref, ift_ref, uwt_ref, iwt_ref, out_ref, ul_acc, il_acc):
    i = pl.program_id(0)
    ul_p = jnp.dot(uwt_ref[0], uft_ref[...], preferred_element_type=jnp.float32)
    il_p = jnp.dot(iwt_ref[0], ift_ref[...], preferred_element_type=jnp.float32)

    @pl.when(i == 0)
    def _init():
        ul_acc[...] = ul_p
        il_acc[...] = il_p

    @pl.when(i > 0)
    def _accum():
        ul_acc[...] += ul_p
        il_acc[...] += il_p

    @pl.when(i == STEPS - 1)
    def _finish():
        ul = ul_acc[...]
        il = il_acc[...]
        out_ref[...] = jnp.sum(ul * il[:L], axis=0) + il[L]


def kernel(user_features, item_features, user_latent_w, item_latent_w, item_biases_w):
    uft = user_features.T
    ift = item_features.T
    uwt = user_latent_w.T.reshape(L, STEPS, KB).transpose(1, 0, 2)
    iwt = (
        jnp.concatenate([item_latent_w.T, item_biases_w.T], axis=0)
        .reshape(L + 1, STEPS, KB)
        .transpose(1, 0, 2)
    )
    return pl.pallas_call(
        _body,
        grid=(STEPS,),
        in_specs=[
            pl.BlockSpec((KB, BATCH), lambda i: (i, 0)),
            pl.BlockSpec((KB, BATCH), lambda i: (i, 0)),
            pl.BlockSpec((1, L, KB), lambda i: (i, 0, 0)),
            pl.BlockSpec((1, L + 1, KB), lambda i: (i, 0, 0)),
        ],
        out_specs=pl.BlockSpec((BATCH,), lambda i: (0,)),
        out_shape=jax.ShapeDtypeStruct((BATCH,), jnp.float32),
        scratch_shapes=[
            pltpu.VMEM((L, BATCH), jnp.float32),
            pltpu.VMEM((L + 1, BATCH), jnp.float32),
        ],
    )(uft, ift, uwt, iwt)


# speedup vs baseline: 1.2031x; 1.1815x over previous
import jax
import jax.numpy as jnp
from jax.experimental import pallas as pl

BATCH = 16384
K = 1000
L = 16
BLK = 1024


def _body(uft_ref, ift_ref, uwt_ref, iwt_ref, out_ref):
    ul = jnp.dot(uwt_ref[...], uft_ref[...], preferred_element_type=jnp.float32)
    il = jnp.dot(iwt_ref[...], ift_ref[...], preferred_element_type=jnp.float32)
    out_ref[...] = jnp.sum(ul * il[:L], axis=0) + il[L]


def kernel(user_features, item_features, user_latent_w, item_latent_w, item_biases_w):
    uft = user_features.T
    ift = item_features.T
    uwt = user_latent_w.T
    iwt = jnp.concatenate([item_latent_w.T, item_biases_w.T], axis=0)
    grid = (BATCH // BLK,)
    return pl.pallas_call(
        _body,
        grid=grid,
        in_specs=[
            pl.BlockSpec((K, BLK), lambda i: (0, i)),
            pl.BlockSpec((K, BLK), lambda i: (0, i)),
            pl.BlockSpec((L, K), lambda i: (0, 0)),
            pl.BlockSpec((L + 1, K), lambda i: (0, 0)),
        ],
        out_specs=pl.BlockSpec((BLK,), lambda i: (i,)),
        out_shape=jax.ShapeDtypeStruct((BATCH,), jnp.float32),
    )(uft, ift, uwt, iwt)
